# Initial kernel scaffold; baseline (speedup 1.0000x reference)
import jax
import jax.numpy as jnp
from jax.experimental import pallas as pl


def kernel(x, edge_index, W1, b1, W2, b2):
    raise NotImplementedError("write your pallas kernel here")


# speedup vs baseline: 17.6967x; 17.6967x over previous
import jax
import jax.numpy as jnp
from jax import lax
from jax.experimental import pallas as pl
from jax.experimental.pallas import tpu as pltpu
from jax.experimental.pallas import tpu_sc as plsc

N_NODES = 10000
N_EDGES = 320000
IN_CH = 128
HID = 64
OUT_CH = 10
CH = 16
K = 3
LAMBDA_AMP = 0.5
GAMMA = 1.0 / (2.0 * (1.0 - LAMBDA_AMP))
G2 = GAMMA * 2.0 * (1.0 - LAMBDA_AMP)
LAM_EFF = GAMMA * LAMBDA_AMP

NC = 2
NS = 16
NW = NC * NS
CHUNK = 128
EPAD = -(-N_EDGES // (NW * CHUNK)) * (NW * CHUNK)
EPT = EPAD // NW
NCHUNK = EPT // CHUNK
NPAD = 10112
SPT = NPAD // NS


def _sc_deg_body(col_hbm, ones_hbm, zeros_hbm, out_hbm, idx_c, msg, acc, sem):
    c = lax.axis_index("c")
    s = lax.axis_index("s")
    w = s * NC + c
    pltpu.sync_copy(ones_hbm, msg)
    pltpu.sync_copy(zeros_hbm.at[pl.ds(s * SPT, SPT)], acc.at[pl.ds(s * SPT, SPT)])
    plsc.subcore_barrier()
    base = w * EPT

    def body(j, carry):
        off = base + j * CHUNK
        pltpu.sync_copy(col_hbm.at[pl.ds(off, CHUNK)], idx_c.at[0])
        pltpu.sync_copy(msg, acc.at[idx_c.at[0]], add=True)
        return carry

    lax.fori_loop(0, NCHUNK, body, 0)
    plsc.subcore_barrier()
    pltpu.sync_copy(acc.at[pl.ds(s * SPT, SPT)],
                    out_hbm.at[c, pl.ds(s * SPT, SPT)])


def _sc_prop_body(u_hbm, row_hbm, col_hbm, zeros_hbm, out_hbm,
                  idx_r, idx_c, msg, acc, sem):
    c = lax.axis_index("c")
    s = lax.axis_index("s")
    w = s * NC + c
    pltpu.sync_copy(zeros_hbm.at[pl.ds(s * SPT, SPT)], acc.at[pl.ds(s * SPT, SPT)])
    plsc.subcore_barrier()
    base = w * EPT

    def body(j, carry):
        off = base + j * CHUNK
        pltpu.sync_copy(row_hbm.at[pl.ds(off, CHUNK)], idx_r.at[0])
        pltpu.sync_copy(col_hbm.at[pl.ds(off, CHUNK)], idx_c.at[0])
        pltpu.async_copy(u_hbm.at[idx_r.at[0]], msg, sem).wait()
        pltpu.sync_copy(msg, acc.at[idx_c.at[0]], add=True)
        return carry

    lax.fori_loop(0, NCHUNK, body, 0)
    plsc.subcore_barrier()
    pltpu.sync_copy(acc.at[pl.ds(s * SPT, SPT)],
                    out_hbm.at[c, pl.ds(s * SPT, SPT)])


_SC_MESH = plsc.VectorSubcoreMesh(core_axis_name="c", subcore_axis_name="s")
_SC_PARAMS = pltpu.CompilerParams(use_tc_tiling_on_sc=False)

_deg_sc = pl.kernel(
    _sc_deg_body,
    out_type=jax.ShapeDtypeStruct((NC, NPAD, CH), jnp.float32),
    mesh=_SC_MESH,
    compiler_params=_SC_PARAMS,
    scratch_types=[
        pltpu.VMEM((1, CHUNK), jnp.int32),
        pltpu.VMEM((CHUNK, CH), jnp.float32),
        pltpu.VMEM_SHARED((NPAD, CH), jnp.float32),
        pltpu.SemaphoreType.DMA,
    ],
)

_prop_sc = pl.kernel(
    _sc_prop_body,
    out_type=jax.ShapeDtypeStruct((NC, NPAD, CH), jnp.float32),
    mesh=_SC_MESH,
    compiler_params=_SC_PARAMS,
    scratch_types=[
        pltpu.VMEM((1, CHUNK), jnp.int32),
        pltpu.VMEM((1, CHUNK), jnp.int32),
        pltpu.VMEM((CHUNK, CH), jnp.float32),
        pltpu.VMEM_SHARED((NPAD, CH), jnp.float32),
        pltpu.SemaphoreType.DMA,
    ],
)


def _mlp_body(x_ref, w1_ref, b1_ref, w2_ref, b2_ref, h_ref):
    h1 = jnp.dot(x_ref[...], w1_ref[...], preferred_element_type=jnp.float32)
    h1 = jnp.maximum(h1 + b1_ref[...], 0.0)
    h_ref[...] = jnp.dot(h1, w2_ref[...],
                         preferred_element_type=jnp.float32) + b2_ref[...]


_mlp = pl.pallas_call(
    _mlp_body,
    out_shape=jax.ShapeDtypeStruct((N_NODES, CH), jnp.float32),
)


def _prep_body(dacc_ref, h_ref, dinv_ref, u_ref):
    dacc = dacc_ref[...]
    deg = 1.0 + dacc[0] + dacc[1]
    dinv = lax.rsqrt(deg)
    dinv_ref[...] = dinv
    u_ref[...] = dinv * h_ref[...]


_prep = pl.pallas_call(
    _prep_body,
    out_shape=(jax.ShapeDtypeStruct((NPAD, CH), jnp.float32),
               jax.ShapeDtypeStruct((NPAD, CH), jnp.float32)),
)


def _step_body(acc_ref, xk_ref, h_ref, dinv_ref, xknew_ref, unew_ref):
    a = acc_ref[...]
    acc = a[0] + a[1]
    dinv = dinv_ref[...]
    xk = xk_ref[...]
    h = h_ref[...]
    y = (1.0 - G2) * xk + G2 * (dinv * acc + dinv * dinv * xk)
    d = y - h
    rn = jnp.sqrt(jnp.sum(d * d, axis=1, keepdims=True))
    scale = jnp.maximum(rn - LAM_EFF, 0.0) / jnp.maximum(rn, 0.5 * LAM_EFF)
    xknew = h + scale * d
    xknew_ref[...] = xknew
    unew_ref[...] = dinv * xknew


_step = pl.pallas_call(
    _step_body,
    out_shape=(jax.ShapeDtypeStruct((NPAD, CH), jnp.float32),
               jax.ShapeDtypeStruct((NPAD, CH), jnp.float32)),
)


def kernel(x, edge_index, W1, b1, W2, b2):
    ei = edge_index.astype(jnp.int32)
    row = jnp.pad(ei[0], (0, EPAD - N_EDGES))
    col = jnp.pad(ei[1], (0, EPAD - N_EDGES), constant_values=N_NODES)
    w2p = jnp.pad(W2, ((0, 0), (0, CH - OUT_CH)))
    b2p = jnp.pad(b2, (0, CH - OUT_CH)).reshape(1, CH)
    b1r = b1.reshape(1, HID)
    zeros = jnp.zeros((NPAD, CH), jnp.float32)
    ones = jnp.ones((CHUNK, CH), jnp.float32)

    h = jnp.pad(_mlp(x, W1, b1r, w2p, b2p), ((0, NPAD - N_NODES), (0, 0)))
    dacc = _deg_sc(col, ones, zeros)
    dinv, u = _prep(dacc, h)
    xk = h
    for _ in range(K):
        acc = _prop_sc(u, row, col, zeros)
        xk, u = _step(acc, xk, h, dinv)
    return xk[:N_NODES, :OUT_CH]


# speedup vs baseline: 31.7677x; 1.7951x over previous
import jax
import jax.numpy as jnp
from jax import lax
from jax.experimental import pallas as pl
from jax.experimental.pallas import tpu as pltpu
from jax.experimental.pallas import tpu_sc as plsc

N_NODES = 10000
N_EDGES = 320000
IN_CH = 128
HID = 64
OUT_CH = 10
CH = 16
K = 3
LAMBDA_AMP = 0.5
GAMMA = 1.0 / (2.0 * (1.0 - LAMBDA_AMP))
G2 = GAMMA * 2.0 * (1.0 - LAMBDA_AMP)
LAM_EFF = GAMMA * LAMBDA_AMP

NC = 2
NS = 16
NW = NC * NS
CHUNK = 128
NCHUNK = 80
NB = 8
NG = NCHUNK // NB
EPT = NCHUNK * CHUNK
EPAD = EPT * NW
NPAD = 10112
SPT = NPAD // NS


def _sc_deg_body(col_hbm, ones_hbm, zeros_hbm, out_hbm, idx_c, msg, acc, ss):
    c = lax.axis_index("c")
    s = lax.axis_index("s")
    w = s * NC + c
    pltpu.sync_copy(ones_hbm, msg)
    pltpu.sync_copy(col_hbm.at[pl.ds(w * NCHUNK, NCHUNK)], idx_c)
    pltpu.sync_copy(zeros_hbm.at[pl.ds(s * SPT, SPT)], acc.at[pl.ds(s * SPT, SPT)])
    plsc.subcore_barrier()

    def body(g, carry):
        descs = [
            pltpu.async_copy(msg, acc.at[idx_c.at[g * NB + b]], ss, add=True)
            for b in range(NB)
        ]
        for d in descs:
            d.wait()
        return carry

    lax.fori_loop(0, NG, body, 0)
    plsc.subcore_barrier()
    pltpu.sync_copy(acc.at[pl.ds(s * SPT, SPT)],
                    out_hbm.at[c, pl.ds(s * SPT, SPT)])


def _sc_prop_body(u_hbm, row_hbm, col_hbm, zeros_hbm, out_hbm,
                  idx_r, idx_c, msg, acc, sg, ss):
    c = lax.axis_index("c")
    s = lax.axis_index("s")
    w = s * NC + c
    pltpu.sync_copy(row_hbm.at[pl.ds(w * NCHUNK, NCHUNK)], idx_r)
    pltpu.sync_copy(col_hbm.at[pl.ds(w * NCHUNK, NCHUNK)], idx_c)
    pltpu.sync_copy(zeros_hbm.at[pl.ds(s * SPT, SPT)], acc.at[pl.ds(s * SPT, SPT)])
    plsc.subcore_barrier()

    def body(g, carry):
        gd = [
            pltpu.async_copy(u_hbm.at[idx_r.at[g * NB + b]], msg.at[b], sg)
            for b in range(NB)
        ]
        for d in gd:
            d.wait()
        sd = [
            pltpu.async_copy(msg.at[b], acc.at[idx_c.at[g * NB + b]], ss, add=True)
            for b in range(NB)
        ]
        for d in sd:
            d.wait()
        return carry

    lax.fori_loop(0, NG, body, 0)
    plsc.subcore_barrier()
    pltpu.sync_copy(acc.at[pl.ds(s * SPT, SPT)],
                    out_hbm.at[c, pl.ds(s * SPT, SPT)])


_SC_MESH = plsc.VectorSubcoreMesh(core_axis_name="c", subcore_axis_name="s")
_SC_PARAMS = pltpu.CompilerParams(use_tc_tiling_on_sc=False)

_deg_sc = pl.kernel(
    _sc_deg_body,
    out_type=jax.ShapeDtypeStruct((NC, NPAD, CH), jnp.float32),
    mesh=_SC_MESH,
    compiler_params=_SC_PARAMS,
    scratch_types=[
        pltpu.VMEM((NCHUNK, CHUNK), jnp.int32),
        pltpu.VMEM((CHUNK, CH), jnp.float32),
        pltpu.VMEM_SHARED((NPAD, CH), jnp.float32),
        pltpu.SemaphoreType.DMA,
    ],
)

_prop_sc = pl.kernel(
    _sc_prop_body,
    out_type=jax.ShapeDtypeStruct((NC, NPAD, CH), jnp.float32),
    mesh=_SC_MESH,
    compiler_params=_SC_PARAMS,
    scratch_types=[
        pltpu.VMEM((NCHUNK, CHUNK), jnp.int32),
        pltpu.VMEM((NCHUNK, CHUNK), jnp.int32),
        pltpu.VMEM((NB, CHUNK, CH), jnp.float32),
        pltpu.VMEM_SHARED((NPAD, CH), jnp.float32),
        pltpu.SemaphoreType.DMA,
        pltpu.SemaphoreType.DMA,
    ],
)


def _mlp_body(x_ref, w1_ref, b1_ref, w2_ref, b2_ref, h_ref):
    h1 = jnp.dot(x_ref[...], w1_ref[...], preferred_element_type=jnp.float32)
    h1 = jnp.maximum(h1 + b1_ref[...], 0.0)
    h_ref[...] = jnp.dot(h1, w2_ref[...],
                         preferred_element_type=jnp.float32) + b2_ref[...]


_mlp = pl.pallas_call(
    _mlp_body,
    out_shape=jax.ShapeDtypeStruct((N_NODES, CH), jnp.float32),
)


def _prep_body(dacc_ref, h_ref, dinv_ref, u_ref):
    dacc = dacc_ref[...]
    deg = 1.0 + dacc[0] + dacc[1]
    dinv = lax.rsqrt(deg)
    dinv_ref[...] = dinv
    u_ref[...] = dinv * h_ref[...]


_prep = pl.pallas_call(
    _prep_body,
    out_shape=(jax.ShapeDtypeStruct((NPAD, CH), jnp.float32),
               jax.ShapeDtypeStruct((NPAD, CH), jnp.float32)),
)


def _step_body(acc_ref, xk_ref, h_ref, dinv_ref, xknew_ref, unew_ref):
    a = acc_ref[...]
    acc = a[0] + a[1]
    dinv = dinv_ref[...]
    xk = xk_ref[...]
    h = h_ref[...]
    y = (1.0 - G2) * xk + G2 * (dinv * acc + dinv * dinv * xk)
    d = y - h
    rn = jnp.sqrt(jnp.sum(d * d, axis=1, keepdims=True))
    scale = jnp.maximum(rn - LAM_EFF, 0.0) / jnp.maximum(rn, 0.5 * LAM_EFF)
    xknew = h + scale * d
    xknew_ref[...] = xknew
    unew_ref[...] = dinv * xknew


_step = pl.pallas_call(
    _step_body,
    out_shape=(jax.ShapeDtypeStruct((NPAD, CH), jnp.float32),
               jax.ShapeDtypeStruct((NPAD, CH), jnp.float32)),
)


def kernel(x, edge_index, W1, b1, W2, b2):
    ei = edge_index.astype(jnp.int32)
    row = jnp.pad(ei[0], (0, EPAD - N_EDGES)).reshape(EPAD // CHUNK, CHUNK)
    col = jnp.pad(ei[1], (0, EPAD - N_EDGES),
                  constant_values=N_NODES).reshape(EPAD // CHUNK, CHUNK)
    w2p = jnp.pad(W2, ((0, 0), (0, CH - OUT_CH)))
    b2p = jnp.pad(b2, (0, CH - OUT_CH)).reshape(1, CH)
    b1r = b1.reshape(1, HID)
    zeros = jnp.zeros((NPAD, CH), jnp.float32)
    ones = jnp.ones((CHUNK, CH), jnp.float32)

    h = jnp.pad(_mlp(x, W1, b1r, w2p, b2p), ((0, NPAD - N_NODES), (0, 0)))
    dacc = _deg_sc(col, ones, zeros)
    dinv, u = _prep(dacc, h)
    xk = h
    for _ in range(K):
        acc = _prop_sc(u, row, col, zeros)
        xk, u = _step(acc, xk, h, dinv)
    return xk[:N_NODES, :OUT_CH]
